# Initial kernel scaffold; baseline (speedup 1.0000x reference)
import jax
import jax.numpy as jnp
from jax.experimental import pallas as pl


def kernel(density_map, keypoints_list, targets_list, domain_pred_rgb, domain_pred_thermal):
    raise NotImplementedError("write your pallas kernel here")


# speedup vs baseline: 3.5926x; 3.5926x over previous
import functools

import jax
import jax.numpy as jnp
from jax import lax
from jax.experimental import pallas as pl
from jax.experimental.pallas import tpu as pltpu
from jax.experimental.pallas import tpu_sc as plsc

B = 16
H = 512
W = 512
N_KP = 2048
LANES = 16
HALF = N_KP // 2
CHUNK = 128
N_CHUNKS = HALF // CHUNK
N_WORKERS = 32


def _sc_body(dens_hbm, kx_hbm, ky_hbm, tg_hbm, out_hbm,
             kx_v, ky_v, tg_v, idx_v, vals_v, acc_v, sem):
    c = lax.axis_index("c")
    s = lax.axis_index("s")
    wid = s * 2 + c
    img = s
    half = c

    cp_kx = pltpu.async_copy(kx_hbm.at[img], kx_v, sem)
    cp_ky = pltpu.async_copy(ky_hbm.at[img], ky_v, sem)
    cp_tg = pltpu.async_copy(tg_hbm.at[img], tg_v, sem)
    cp_kx.wait()
    cp_ky.wait()
    cp_tg.wait()

    def _max_step(j, carry):
        mx, my = carry
        xv = kx_v[pl.ds(j * LANES, LANES)]
        yv = ky_v[pl.ds(j * LANES, LANES)]
        return jnp.maximum(mx, xv), jnp.maximum(my, yv)

    mx0 = kx_v[pl.ds(0, LANES)]
    my0 = ky_v[pl.ds(0, LANES)]
    mx, my = lax.fori_loop(1, N_KP // LANES, _max_step, (mx0, my0))

    def _lane_max(vec):
        m = vec[0]
        for k in range(1, LANES):
            m = jnp.maximum(m, vec[k])
        return m

    max_x = _lane_max(mx)
    max_y = _lane_max(my)

    def _scale(mval, dim):
        mvec = lax.broadcast(mval, (LANES,))
        s = jnp.full((LANES,), jnp.float32(dim)) / mvec
        return jnp.where(mvec > 0, s, jnp.full((LANES,), jnp.float32(1.0)))

    scale_w = _scale(max_x, W)
    scale_h = _scale(max_y, H)

    base_pt = half * HALF
    img_off = img * (H * W)

    for j in range(N_CHUNKS):
        for v in range(CHUNK // LANES):
            off = base_pt + j * CHUNK + v * LANES
            xv = kx_v[pl.ds(off, LANES)]
            yv = ky_v[pl.ds(off, LANES)]
            ix = jnp.clip((xv * scale_w).astype(jnp.int32), 0, W - 1)
            iy = jnp.clip((yv * scale_h).astype(jnp.int32), 0, H - 1)
            idx_v[j, pl.ds(v * LANES, LANES)] = img_off + iy * W + ix

    gathers = [
        pltpu.async_copy(dens_hbm.at[idx_v.at[j]], vals_v.at[j], sem)
        for j in range(N_CHUNKS)
    ]
    for g in gathers:
        g.wait()

    acc = jnp.zeros((LANES,), jnp.float32)
    for j in range(N_CHUNKS):
        for v in range(CHUNK // LANES):
            pv = vals_v[j, pl.ds(v * LANES, LANES)]
            tv = tg_v[pl.ds(base_pt + j * CHUNK + v * LANES, LANES)]
            d = pv - tv
            acc = acc + d * d
    acc_v[...] = acc
    pltpu.async_copy(acc_v, out_hbm.at[wid], sem).wait()


def _tc_finalize_body(part_ref, rgb_ref, th_ref, out_ref):
    alpha = jnp.float32(0.1)
    density_loss = jnp.sum(part_ref[...]) / jnp.float32(B * N_KP)
    lp_rgb = jax.nn.log_softmax(rgb_ref[...], axis=-1)
    lp_th = jax.nn.log_softmax(th_ref[...], axis=-1)
    ce_rgb = -jnp.mean(lp_rgb[:, 0])
    ce_th = -jnp.mean(lp_th[:, 1])
    domain_loss = (ce_rgb + ce_th) * jnp.float32(0.5)
    out_ref[0] = density_loss + alpha * domain_loss
    out_ref[1] = density_loss
    out_ref[2] = domain_loss


def kernel(density_map, keypoints_list, targets_list,
           domain_pred_rgb, domain_pred_thermal):
    dens_flat = density_map.reshape(B * H * W)
    kx = keypoints_list[:, :, 0]
    ky = keypoints_list[:, :, 1]

    mesh = plsc.VectorSubcoreMesh(core_axis_name="c", subcore_axis_name="s")
    sc_kernel = pl.kernel(
        _sc_body,
        out_type=jax.ShapeDtypeStruct((N_WORKERS, LANES), jnp.float32),
        mesh=mesh,
        scratch_types=[
            pltpu.VMEM((N_KP,), jnp.float32),
            pltpu.VMEM((N_KP,), jnp.float32),
            pltpu.VMEM((N_KP,), jnp.float32),
            pltpu.VMEM((N_CHUNKS, CHUNK), jnp.int32),
            pltpu.VMEM((N_CHUNKS, CHUNK), jnp.float32),
            pltpu.VMEM((LANES,), jnp.float32),
            pltpu.SemaphoreType.DMA,
        ],
    )
    partials = sc_kernel(dens_flat, kx, ky, targets_list)

    out = pl.pallas_call(
        _tc_finalize_body,
        out_shape=jax.ShapeDtypeStruct((3,), jnp.float32),
        out_specs=pl.BlockSpec(memory_space=pltpu.SMEM),
    )(partials, domain_pred_rgb, domain_pred_thermal)

    return (out[0], out[1], out[2])


# speedup vs baseline: 5.0974x; 1.4189x over previous
import jax
import jax.numpy as jnp
from jax import lax
from jax.experimental import pallas as pl
from jax.experimental.pallas import tpu as pltpu
from jax.experimental.pallas import tpu_sc as plsc

B = 16
H = 512
W = 512
N_KP = 2048
LANES = 16
BAND = 128
N_PASS = 2
N_WORKERS = 32


def _sc_body(dens_hbm, kx_hbm, ky_hbm, tg_hbm, out_hbm,
             kx_v, ky_v, tg_v, band_v, acc_v, sem):
    dens2 = dens_hbm
    c = lax.axis_index("c")
    s = lax.axis_index("s")
    wid = s * 2 + c
    img = s

    cp_kx = pltpu.async_copy(kx_hbm.at[img], kx_v, sem)
    cp_ky = pltpu.async_copy(ky_hbm.at[img], ky_v, sem)
    cp_tg = pltpu.async_copy(tg_hbm.at[img], tg_v, sem)
    cp_kx.wait()
    cp_ky.wait()
    cp_tg.wait()

    def _max_step(j, carry):
        mx, my = carry
        xv = kx_v[pl.ds(j * LANES, LANES)]
        yv = ky_v[pl.ds(j * LANES, LANES)]
        return jnp.maximum(mx, xv), jnp.maximum(my, yv)

    mx0 = kx_v[pl.ds(0, LANES)]
    my0 = ky_v[pl.ds(0, LANES)]
    mx, my = lax.fori_loop(1, N_KP // LANES, _max_step, (mx0, my0))

    def _lane_max(vec):
        m = vec[0]
        for k in range(1, LANES):
            m = jnp.maximum(m, vec[k])
        return m

    max_x = _lane_max(mx)
    max_y = _lane_max(my)

    def _scale(mval, dim):
        mvec = lax.broadcast(mval, (LANES,))
        sc = jnp.full((LANES,), jnp.float32(dim)) / mvec
        return jnp.where(mvec > 0, sc, jnp.full((LANES,), jnp.float32(1.0)))

    scale_w = _scale(max_x, W)
    scale_h = _scale(max_y, H)

    acc = jnp.zeros((LANES,), jnp.float32)
    for p in range(N_PASS):
        y0 = c * (N_PASS * BAND) + p * BAND
        pltpu.async_copy(dens2.at[pl.ds(img * H + y0, BAND)], band_v,
                         sem).wait()

        def _pass_step(j, acc, y0=y0):
            xv = kx_v[pl.ds(j * LANES, LANES)]
            yv = ky_v[pl.ds(j * LANES, LANES)]
            tv = tg_v[pl.ds(j * LANES, LANES)]
            ix = jnp.clip((xv * scale_w).astype(jnp.int32), 0, W - 1)
            iy = jnp.clip((yv * scale_h).astype(jnp.int32), 0, H - 1)
            t = iy - y0
            m = (t >= 0) & (t < BAND)
            pv = plsc.load_gather(band_v, [t, ix], mask=m)
            d = jnp.where(m, pv - tv, jnp.float32(0.0))
            return acc + d * d

        acc = lax.fori_loop(0, N_KP // LANES, _pass_step, acc)

    acc_v[...] = acc
    pltpu.async_copy(acc_v, out_hbm.at[wid], sem).wait()


def _tc_finalize_body(part_ref, rgb_ref, th_ref, out_ref):
    alpha = jnp.float32(0.1)
    density_loss = jnp.sum(part_ref[...]) / jnp.float32(B * N_KP)
    lp_rgb = jax.nn.log_softmax(rgb_ref[...], axis=-1)
    lp_th = jax.nn.log_softmax(th_ref[...], axis=-1)
    ce_rgb = -jnp.mean(lp_rgb[:, 0])
    ce_th = -jnp.mean(lp_th[:, 1])
    domain_loss = (ce_rgb + ce_th) * jnp.float32(0.5)
    out_ref[0] = density_loss + alpha * domain_loss
    out_ref[1] = density_loss
    out_ref[2] = domain_loss


def kernel(density_map, keypoints_list, targets_list,
           domain_pred_rgb, domain_pred_thermal):
    kx = keypoints_list[:, :, 0]
    ky = keypoints_list[:, :, 1]

    mesh = plsc.VectorSubcoreMesh(core_axis_name="c", subcore_axis_name="s")
    sc_kernel = pl.kernel(
        _sc_body,
        out_type=jax.ShapeDtypeStruct((N_WORKERS, LANES), jnp.float32),
        mesh=mesh,
        scratch_types=[
            pltpu.VMEM((N_KP,), jnp.float32),
            pltpu.VMEM((N_KP,), jnp.float32),
            pltpu.VMEM((N_KP,), jnp.float32),
            pltpu.VMEM((BAND, W), jnp.float32),
            pltpu.VMEM((LANES,), jnp.float32),
            pltpu.SemaphoreType.DMA,
        ],
        compiler_params=pltpu.CompilerParams(needs_layout_passes=False),
    )
    partials = sc_kernel(density_map.reshape(B * H, W), kx, ky, targets_list)

    out = pl.pallas_call(
        _tc_finalize_body,
        out_shape=jax.ShapeDtypeStruct((3,), jnp.float32),
        out_specs=pl.BlockSpec(memory_space=pltpu.SMEM),
    )(partials, domain_pred_rgb, domain_pred_thermal)

    return (out[0], out[1], out[2])


# speedup vs baseline: 5.2994x; 1.0396x over previous
import jax
import jax.numpy as jnp
from jax import lax
from jax.experimental import pallas as pl
from jax.experimental.pallas import tpu as pltpu
from jax.experimental.pallas import tpu_sc as plsc

B = 16
H = 512
W = 512
N_KP = 2048
LANES = 16
BAND = 128
N_PASS = 2
N_WORKERS = 32


def _sc_body(dens_hbm, kx_hbm, ky_hbm, tg_hbm, out_hbm,
             kx_v, ky_v, tg_v, band_v, acc_v, sem, sem_band):
    dens2 = dens_hbm
    c = lax.axis_index("c")
    s = lax.axis_index("s")
    wid = s * 2 + c
    img = s
    ybase = c * (N_PASS * BAND)

    cp_b0 = pltpu.async_copy(dens2.at[pl.ds(img * H + ybase, BAND)],
                             band_v, sem_band)
    cp_kx = pltpu.async_copy(kx_hbm.at[img], kx_v, sem)
    cp_ky = pltpu.async_copy(ky_hbm.at[img], ky_v, sem)
    cp_tg = pltpu.async_copy(tg_hbm.at[img], tg_v, sem)
    cp_kx.wait()
    cp_ky.wait()
    cp_tg.wait()

    def _max_step(j, carry):
        mx, my = carry
        xv = kx_v[pl.ds(j * LANES, LANES)]
        yv = ky_v[pl.ds(j * LANES, LANES)]
        return jnp.maximum(mx, xv), jnp.maximum(my, yv)

    mx0 = kx_v[pl.ds(0, LANES)]
    my0 = ky_v[pl.ds(0, LANES)]
    mx, my = lax.fori_loop(1, N_KP // LANES, _max_step, (mx0, my0))

    max_x = jnp.max(mx)
    max_y = jnp.max(my)

    def _scale(mval, dim):
        mvec = lax.broadcast(mval, (LANES,))
        sc = jnp.full((LANES,), jnp.float32(dim)) / mvec
        return jnp.where(mvec > 0, sc, jnp.full((LANES,), jnp.float32(1.0)))

    scale_w = _scale(max_x, W)
    scale_h = _scale(max_y, H)

    acc = jnp.zeros((LANES,), jnp.float32)
    for p in range(N_PASS):
        y0 = ybase + p * BAND
        if p == 0:
            cp_b0.wait()
        else:
            pltpu.async_copy(dens2.at[pl.ds(img * H + y0, BAND)], band_v,
                             sem_band).wait()

        def _pass_step(j, acc, y0=y0):
            xv = kx_v[pl.ds(j * LANES, LANES)]
            yv = ky_v[pl.ds(j * LANES, LANES)]
            tv = tg_v[pl.ds(j * LANES, LANES)]
            ix = jnp.clip((xv * scale_w).astype(jnp.int32), 0, W - 1)
            iy = jnp.clip((yv * scale_h).astype(jnp.int32), 0, H - 1)
            t = iy - y0
            m = (t >= 0) & (t < BAND)
            pv = plsc.load_gather(band_v, [t, ix], mask=m)
            d = jnp.where(m, pv - tv, jnp.float32(0.0))
            return acc + d * d

        acc = lax.fori_loop(0, N_KP // LANES, _pass_step, acc)

    acc_v[...] = acc
    pltpu.async_copy(acc_v, out_hbm.at[wid], sem).wait()


def _tc_finalize_body(part_ref, rgb_ref, th_ref, out_ref):
    alpha = jnp.float32(0.1)
    density_loss = jnp.sum(part_ref[...]) / jnp.float32(B * N_KP)
    lp_rgb = jax.nn.log_softmax(rgb_ref[...], axis=-1)
    lp_th = jax.nn.log_softmax(th_ref[...], axis=-1)
    ce_rgb = -jnp.mean(lp_rgb[:, 0])
    ce_th = -jnp.mean(lp_th[:, 1])
    domain_loss = (ce_rgb + ce_th) * jnp.float32(0.5)
    out_ref[0] = density_loss + alpha * domain_loss
    out_ref[1] = density_loss
    out_ref[2] = domain_loss


def kernel(density_map, keypoints_list, targets_list,
           domain_pred_rgb, domain_pred_thermal):
    kx = keypoints_list[:, :, 0]
    ky = keypoints_list[:, :, 1]

    mesh = plsc.VectorSubcoreMesh(core_axis_name="c", subcore_axis_name="s")
    sc_kernel = pl.kernel(
        _sc_body,
        out_type=jax.ShapeDtypeStruct((N_WORKERS, LANES), jnp.float32),
        mesh=mesh,
        scratch_types=[
            pltpu.VMEM((N_KP,), jnp.float32),
            pltpu.VMEM((N_KP,), jnp.float32),
            pltpu.VMEM((N_KP,), jnp.float32),
            pltpu.VMEM((BAND, W), jnp.float32),
            pltpu.VMEM((LANES,), jnp.float32),
            pltpu.SemaphoreType.DMA,
            pltpu.SemaphoreType.DMA,
        ],
        compiler_params=pltpu.CompilerParams(needs_layout_passes=False),
    )
    partials = sc_kernel(density_map.reshape(B * H, W), kx, ky, targets_list)

    out = pl.pallas_call(
        _tc_finalize_body,
        out_shape=jax.ShapeDtypeStruct((3,), jnp.float32),
        out_specs=pl.BlockSpec(memory_space=pltpu.SMEM),
    )(partials, domain_pred_rgb, domain_pred_thermal)

    return (out[0], out[1], out[2])
